# Initial kernel scaffold; baseline (speedup 1.0000x reference)
import jax
import jax.numpy as jnp
from jax.experimental import pallas as pl


def kernel(x, position_weight, level_weight):
    raise NotImplementedError("write your pallas kernel here")


# speedup vs baseline: 5.3725x; 5.3725x over previous
import functools

import jax
import jax.numpy as jnp
from jax.experimental import pallas as pl
from jax.experimental.pallas import tpu as pltpu

_B = 8
_N = 784
_L = 256
_D = 10000
_DT = 512


def _body(x_ref, p_ref, lw_ref, o_ref, oh_ref):
    @pl.when(pl.program_id(0) == 0)
    def _():
        flat = x_ref[...]
        idx = jnp.clip(jnp.round(flat * (_L - 1)), 0, _L - 1).astype(jnp.int32)
        lvl = jax.lax.broadcasted_iota(jnp.int32, (_L, _N), 0)
        for b in range(_B):
            ohT = (lvl == idx[b][None, :]).astype(jnp.bfloat16)
            oh_ref[b] = ohT

    p_bf = p_ref[...].astype(jnp.bfloat16)
    lw = lw_ref[...]
    for b in range(_B):
        a = jax.lax.dot(oh_ref[b], p_bf, preferred_element_type=jnp.float32)
        ms = jnp.sum(a * lw, axis=0)
        o_ref[b, :] = jnp.where(ms > 0, jnp.float32(1.0), jnp.float32(-1.0))


@jax.jit
def kernel(x, position_weight, level_weight):
    flat = x.reshape(_B, _N)
    grid = (pl.cdiv(_D, _DT),)
    return pl.pallas_call(
        _body,
        grid=grid,
        in_specs=[
            pl.BlockSpec((_B, _N), lambda j: (0, 0)),
            pl.BlockSpec((_N, _DT), lambda j: (0, j)),
            pl.BlockSpec((_L, _DT), lambda j: (0, j)),
        ],
        out_specs=pl.BlockSpec((_B, _DT), lambda j: (0, j)),
        out_shape=jax.ShapeDtypeStruct((_B, _D), jnp.float32),
        scratch_shapes=[pltpu.VMEM((_B, _L, _N), jnp.bfloat16)],
    )(flat, position_weight, level_weight)
